# baseline (speedup 1.0000x reference)
import functools

import jax
import jax.numpy as jnp
from jax import lax
from jax.experimental import pallas as pl
from jax.experimental.pallas import tpu as pltpu
from jax.experimental.pallas import tpu_sc as plsc

_B, _T, _VOCAB = 32, 16, 100000
_TEMP = 0.8
_REP = 1.2
_BASE = 1.0 / _TEMP
_PEN = 1.0 / (_TEMP * _REP)

_NC, _NS, _L = 2, 16, 16
_NW = _NC * _NS
_NIDS = _B * _T
_CHUNK = 3200
_VPAD = _NW * _CHUNK

_sc_mesh = plsc.VectorSubcoreMesh(core_axis_name="c", subcore_axis_name="s")


@functools.partial(
    pl.kernel,
    mesh=_sc_mesh,
    out_type=jax.ShapeDtypeStruct((_VPAD,), jnp.float32),
    scratch_types=[
        pltpu.VMEM((_B, _T), jnp.int32),
        pltpu.VMEM((_CHUNK,), jnp.float32),
    ],
    compiler_params=pltpu.CompilerParams(needs_layout_passes=False),
)
def _sc_build_scale(ids_hbm, out_hbm, ids_v, chunk_v):
    wid = lax.axis_index("s") * _NC + lax.axis_index("c")
    start = wid * _CHUNK
    pltpu.sync_copy(ids_hbm, ids_v)
    base = jnp.full((_L,), _BASE, jnp.float32)

    def fill(i, _):
        chunk_v[pl.ds(i * _L, _L)] = base
        return 0
    lax.fori_loop(0, _CHUNK // _L, fill, 0)

    pen = jnp.full((_L,), _PEN, jnp.float32)

    def scat(j, _):
        ids = ids_v[j, :]
        loc = ids - start
        msk = (loc >= 0) & (loc < _CHUNK)
        loc = jnp.where(msk, loc, 0)
        plsc.store_scatter(chunk_v, [loc], pen, mask=msk)
        return 0
    lax.fori_loop(0, _B, scat, 0)
    pltpu.sync_copy(chunk_v, out_hbm.at[pl.ds(start, _CHUNK)])


_RB = 32


def _tc_body(s_ref, x_ref, o_ref):
    o_ref[...] = x_ref[...] * s_ref[:, : _VOCAB]


def _tc_apply_full(x2d, scale2d):
    return pl.pallas_call(
        _tc_body,
        grid=(x2d.shape[0] // _RB,),
        in_specs=[
            pl.BlockSpec((1, _VPAD), lambda i: (0, 0)),
            pl.BlockSpec((_RB, _VOCAB), lambda i: (i, 0)),
        ],
        out_specs=pl.BlockSpec((_RB, _VOCAB), lambda i: (i, 0)),
        out_shape=jax.ShapeDtypeStruct(x2d.shape, jnp.float32),
    )(scale2d, x2d)


def kernel(logits, input_ids):
    scale = _sc_build_scale(input_ids.astype(jnp.int32))
    scale2d = scale.reshape(1, _VPAD)
    x2d = logits.reshape(_B * _T, _VOCAB)
    out = _tc_apply_full(x2d, scale2d)
    return out.reshape(_B, _T, _VOCAB)


# speedup vs baseline: 1.0059x; 1.0059x over previous
import functools

import jax
import jax.numpy as jnp
from jax import lax
from jax.experimental import pallas as pl
from jax.experimental.pallas import tpu as pltpu
from jax.experimental.pallas import tpu_sc as plsc

_B, _T, _VOCAB = 32, 16, 100000
_TEMP = 0.8
_REP = 1.2
_BASE = 1.0 / _TEMP
_PEN = 1.0 / (_TEMP * _REP)

_NC, _NS, _L = 1, 16, 16
_NW = _NC * _NS
_NIDS = _B * _T
_CHUNK = 6400
_VPAD = _NW * _CHUNK

_sc_mesh = plsc.VectorSubcoreMesh(
    core_axis_name="c", subcore_axis_name="s", num_cores=_NC)


@functools.partial(
    pl.kernel,
    mesh=_sc_mesh,
    out_type=jax.ShapeDtypeStruct((_VPAD,), jnp.float32),
    scratch_types=[
        pltpu.VMEM((_B, _T), jnp.int32),
        pltpu.VMEM((_CHUNK,), jnp.float32),
    ],
    compiler_params=pltpu.CompilerParams(needs_layout_passes=False),
)
def _sc_build_scale(ids_hbm, out_hbm, ids_v, chunk_v):
    wid = lax.axis_index("s") * _NC + lax.axis_index("c")
    start = wid * _CHUNK
    pltpu.sync_copy(ids_hbm, ids_v)
    base = jnp.full((_L,), _BASE, jnp.float32)

    def fill(i, _):
        chunk_v[pl.ds(i * _L, _L)] = base
        return 0
    lax.fori_loop(0, _CHUNK // _L, fill, 0)

    pen = jnp.full((_L,), _PEN, jnp.float32)

    def scat(j, _):
        ids = ids_v[j, :]
        loc = ids - start
        msk = (loc >= 0) & (loc < _CHUNK)
        loc = jnp.where(msk, loc, 0)
        plsc.store_scatter(chunk_v, [loc], pen, mask=msk)
        return 0
    lax.fori_loop(0, _B, scat, 0)
    pltpu.sync_copy(chunk_v, out_hbm.at[pl.ds(start, _CHUNK)])


_RB = 32


def _tc_body(s_ref, x_ref, o_ref):
    o_ref[...] = x_ref[...] * s_ref[:, : _VOCAB]


def _tc_apply_full(x2d, scale2d):
    return pl.pallas_call(
        _tc_body,
        grid=(x2d.shape[0] // _RB,),
        in_specs=[
            pl.BlockSpec((1, _VPAD), lambda i: (0, 0)),
            pl.BlockSpec((_RB, _VOCAB), lambda i: (i, 0)),
        ],
        out_specs=pl.BlockSpec((_RB, _VOCAB), lambda i: (i, 0)),
        out_shape=jax.ShapeDtypeStruct(x2d.shape, jnp.float32),
    )(scale2d, x2d)


def kernel(logits, input_ids):
    scale = _sc_build_scale(input_ids.astype(jnp.int32))
    scale2d = scale.reshape(1, _VPAD)
    x2d = logits.reshape(_B * _T, _VOCAB)
    out = _tc_apply_full(x2d, scale2d)
    return out.reshape(_B, _T, _VOCAB)


# speedup vs baseline: 1.0173x; 1.0113x over previous
import functools

import jax
import jax.numpy as jnp
from jax import lax
from jax.experimental import pallas as pl
from jax.experimental.pallas import tpu as pltpu
from jax.experimental.pallas import tpu_sc as plsc

_B, _T, _VOCAB = 32, 16, 100000
_TEMP = 0.8
_REP = 1.2
_BASE = 1.0 / _TEMP
_PEN = 1.0 / (_TEMP * _REP)

_NC, _NS, _L = 1, 16, 16
_NW = _NC * _NS
_NIDS = _B * _T
_CHUNK = 6400
_VPAD = _NW * _CHUNK

_sc_mesh = plsc.VectorSubcoreMesh(
    core_axis_name="c", subcore_axis_name="s", num_cores=_NC)


@functools.partial(
    pl.kernel,
    mesh=_sc_mesh,
    out_type=jax.ShapeDtypeStruct((_VPAD,), jnp.float32),
    scratch_types=[
        pltpu.VMEM((_B, _T), jnp.int32),
        pltpu.VMEM((_CHUNK,), jnp.float32),
    ],
    compiler_params=pltpu.CompilerParams(needs_layout_passes=False),
)
def _sc_build_scale(ids_hbm, out_hbm, ids_v, chunk_v):
    wid = lax.axis_index("s") * _NC + lax.axis_index("c")
    start = wid * _CHUNK
    pltpu.sync_copy(ids_hbm, ids_v)
    base = jnp.full((_L,), _BASE, jnp.float32)

    def fill(i, _):
        for u in range(8):
            chunk_v[pl.ds((i * 8 + u) * _L, _L)] = base
        return 0
    lax.fori_loop(0, _CHUNK // (8 * _L), fill, 0)

    pen = jnp.full((_L,), _PEN, jnp.float32)

    def scat(j, _):
        ids = ids_v[j, :]
        loc = ids - start
        msk = (loc >= 0) & (loc < _CHUNK)
        loc = jnp.where(msk, loc, 0)
        plsc.store_scatter(chunk_v, [loc], pen, mask=msk)
        return 0
    lax.fori_loop(0, _B, scat, 0)
    pltpu.sync_copy(chunk_v, out_hbm.at[pl.ds(start, _CHUNK)])


_RB = 32


def _tc_body(s_ref, x_ref, o_ref):
    o_ref[...] = x_ref[...] * s_ref[:, : _VOCAB]


def _tc_apply_full(x2d, scale2d):
    return pl.pallas_call(
        _tc_body,
        grid=(x2d.shape[0] // _RB,),
        in_specs=[
            pl.BlockSpec((1, _VPAD), lambda i: (0, 0)),
            pl.BlockSpec((_RB, _VOCAB), lambda i: (i, 0)),
        ],
        out_specs=pl.BlockSpec((_RB, _VOCAB), lambda i: (i, 0)),
        out_shape=jax.ShapeDtypeStruct(x2d.shape, jnp.float32),
    )(scale2d, x2d)


def kernel(logits, input_ids):
    scale = _sc_build_scale(input_ids.astype(jnp.int32))
    scale2d = scale.reshape(1, _VPAD)
    x2d = logits.reshape(_B * _T, _VOCAB)
    out = _tc_apply_full(x2d, scale2d)
    return out.reshape(_B, _T, _VOCAB)
